# Initial kernel scaffold; baseline (speedup 1.0000x reference)
import jax
import jax.numpy as jnp
from jax.experimental import pallas as pl


def kernel(prob_vec):
    raise NotImplementedError("write your pallas kernel here")


# speedup vs baseline: 1.2185x; 1.2185x over previous
import functools

import jax
import jax.numpy as jnp
from jax import lax
from jax.experimental import pallas as pl
from jax.experimental.pallas import tpu as pltpu


def _rowmax_kernel(x_ref, id_ref, max_ref):
    x = x_ref[...]
    m = jnp.max(x, axis=1, keepdims=True)
    n = x.shape[1]
    iota = lax.broadcasted_iota(jnp.int32, x.shape, 1)
    cand = jnp.where(x == m, iota, jnp.int32(n))
    idx = jnp.min(cand, axis=1, keepdims=True)
    id_ref[...] = idx.astype(jnp.float32)
    max_ref[...] = m


@functools.partial(jax.jit, static_argnames=("block_rows",))
def _rowmax(x2d, block_rows=32):
    rows, n = x2d.shape
    grid = (rows // block_rows,)
    id_out, max_out = pl.pallas_call(
        _rowmax_kernel,
        grid=grid,
        in_specs=[pl.BlockSpec((block_rows, n), lambda i: (i, 0))],
        out_specs=[
            pl.BlockSpec((block_rows, 1), lambda i: (i, 0)),
            pl.BlockSpec((block_rows, 1), lambda i: (i, 0)),
        ],
        out_shape=[
            jax.ShapeDtypeStruct((rows, 1), jnp.float32),
            jax.ShapeDtypeStruct((rows, 1), jnp.float32),
        ],
        compiler_params=pltpu.CompilerParams(
            dimension_semantics=("arbitrary",),
        ),
    )(x2d)
    return id_out, max_out


def kernel(prob_vec):
    b, s, n = prob_vec.shape
    x2d = prob_vec.reshape(b * s, n)
    id_out, max_out = _rowmax(x2d)
    out = jnp.concatenate([id_out, max_out], axis=1)
    return out.reshape(b, s, 2)


# speedup vs baseline: 1.4786x; 1.2134x over previous
import functools

import jax
import jax.numpy as jnp
from jax import lax
from jax.experimental import pallas as pl
from jax.experimental.pallas import tpu as pltpu


def _rowmax_kernel(x_ref, id_ref, max_ref):
    x = x_ref[...]
    m = jnp.max(x, axis=1, keepdims=True)
    n = x.shape[1]
    iota = lax.broadcasted_iota(jnp.int32, x.shape, 1)
    cand = jnp.where(x == m, iota, jnp.int32(n))
    idx = jnp.min(cand, axis=1, keepdims=True)
    id_ref[...] = idx.astype(jnp.float32)
    max_ref[...] = m


@functools.partial(jax.jit, static_argnames=("block_rows",))
def _rowmax(x2d, block_rows=64):
    rows, n = x2d.shape
    grid = (rows // block_rows,)
    id_out, max_out = pl.pallas_call(
        _rowmax_kernel,
        grid=grid,
        in_specs=[pl.BlockSpec((block_rows, n), lambda i: (i, 0))],
        out_specs=[
            pl.BlockSpec((block_rows, 1), lambda i: (i, 0)),
            pl.BlockSpec((block_rows, 1), lambda i: (i, 0)),
        ],
        out_shape=[
            jax.ShapeDtypeStruct((rows, 1), jnp.float32),
            jax.ShapeDtypeStruct((rows, 1), jnp.float32),
        ],
        compiler_params=pltpu.CompilerParams(
            dimension_semantics=("arbitrary",),
        ),
    )(x2d)
    return id_out, max_out


def kernel(prob_vec):
    b, s, n = prob_vec.shape
    x2d = prob_vec.reshape(b * s, n)
    id_out, max_out = _rowmax(x2d)
    out = jnp.concatenate([id_out, max_out], axis=1)
    return out.reshape(b, s, 2)


# speedup vs baseline: 1.5654x; 1.0588x over previous
import functools

import jax
import jax.numpy as jnp
from jax import lax
from jax.experimental import pallas as pl
from jax.experimental.pallas import tpu as pltpu


def _rowmax_kernel(x_ref, id_ref, max_ref):
    x = x_ref[...]
    m = jnp.max(x, axis=1, keepdims=True)
    n = x.shape[1]
    iota = lax.broadcasted_iota(jnp.int32, x.shape, 1)
    cand = jnp.where(x == m, iota, jnp.int32(n))
    idx = jnp.min(cand, axis=1, keepdims=True)
    id_ref[...] = idx.astype(jnp.float32)
    max_ref[...] = m


@functools.partial(jax.jit, static_argnames=("block_rows",))
def _rowmax(x2d, block_rows=128):
    rows, n = x2d.shape
    grid = (rows // block_rows,)
    id_out, max_out = pl.pallas_call(
        _rowmax_kernel,
        grid=grid,
        in_specs=[pl.BlockSpec((block_rows, n), lambda i: (i, 0))],
        out_specs=[
            pl.BlockSpec((block_rows, 1), lambda i: (i, 0)),
            pl.BlockSpec((block_rows, 1), lambda i: (i, 0)),
        ],
        out_shape=[
            jax.ShapeDtypeStruct((rows, 1), jnp.float32),
            jax.ShapeDtypeStruct((rows, 1), jnp.float32),
        ],
        compiler_params=pltpu.CompilerParams(
            dimension_semantics=("arbitrary",),
        ),
    )(x2d)
    return id_out, max_out


def kernel(prob_vec):
    b, s, n = prob_vec.shape
    x2d = prob_vec.reshape(b * s, n)
    id_out, max_out = _rowmax(x2d)
    out = jnp.concatenate([id_out, max_out], axis=1)
    return out.reshape(b, s, 2)
